# baseline (speedup 1.0000x reference)
import jax
import jax.numpy as jnp
from jax.experimental import pallas as pl
from jax.experimental.pallas import tpu as pltpu

_EDGE_IN = 19
_CHUNKS = 10


def _edge_pred_kernel(w_ref, b_ref, et_ref, o_hbm, obuf, sems):
    n_edges = et_ref.shape[1]
    chunk = n_edges // _CHUNKS
    w = w_ref[...].astype(jnp.bfloat16)
    b = b_ref[...]
    for i in range(_CHUNKS):
        sl = pl.ds(i * chunk, chunk)
        obuf[:, sl] = (
            jnp.dot(
                w,
                et_ref[:, sl].astype(jnp.bfloat16),
                preferred_element_type=jnp.float32,
            )
            + b
        )
        pltpu.make_async_copy(
            obuf.at[:, sl], o_hbm.at[:, sl], sems.at[i]
        ).start()
    for i in range(_CHUNKS):
        pltpu.make_async_copy(
            obuf.at[:, pl.ds(i * chunk, chunk)],
            o_hbm.at[:, pl.ds(i * chunk, chunk)],
            sems.at[i],
        ).wait()


def kernel(x, edge_index, e, xbatch, bn_g0, bn_b0, W00, b00, W01, b01,
           root0, rb0, bn_g1, bn_b1, W10, b10, W11, b11, root1, rb1,
           bn_g2, bn_b2, W20, b20, W21, b21, root2, rb2, Wp, bp):
    e = e.reshape(-1, _EDGE_IN)
    n_edges = e.shape[0]
    n_out = Wp.shape[1]
    et = e.T
    wt = Wp.T
    bias = bp.reshape(n_out, 1)

    out_t = pl.pallas_call(
        _edge_pred_kernel,
        in_specs=[
            pl.BlockSpec(memory_space=pltpu.VMEM),
            pl.BlockSpec(memory_space=pltpu.VMEM),
            pl.BlockSpec(memory_space=pltpu.VMEM),
        ],
        out_specs=pl.BlockSpec(memory_space=pl.ANY),
        out_shape=jax.ShapeDtypeStruct((n_out, n_edges), jnp.float32),
        scratch_shapes=[
            pltpu.VMEM((n_out, n_edges), jnp.float32),
            pltpu.SemaphoreType.DMA((_CHUNKS,)),
        ],
    )(wt, bias, et)
    return out_t.T


# speedup vs baseline: 1.0457x; 1.0457x over previous
import jax
import jax.numpy as jnp
from jax.experimental import pallas as pl
from jax.experimental.pallas import tpu as pltpu

_EDGE_IN = 19
_CHUNKS = 10


def _edge_pred_kernel(w_ref, b_ref, et_ref, o_hbm, obuf, sems):
    n_edges = et_ref.shape[1]
    chunk = n_edges // _CHUNKS
    w = w_ref[...]
    b = b_ref[...]
    for i in range(_CHUNKS):
        sl = pl.ds(i * chunk, chunk)
        obuf[:, sl] = (
            jnp.dot(w, et_ref[:, sl], preferred_element_type=jnp.float32) + b
        )
        pltpu.make_async_copy(
            obuf.at[:, sl], o_hbm.at[:, sl], sems.at[i]
        ).start()
    for i in range(_CHUNKS):
        pltpu.make_async_copy(
            obuf.at[:, pl.ds(i * chunk, chunk)],
            o_hbm.at[:, pl.ds(i * chunk, chunk)],
            sems.at[i],
        ).wait()


def kernel(x, edge_index, e, xbatch, bn_g0, bn_b0, W00, b00, W01, b01,
           root0, rb0, bn_g1, bn_b1, W10, b10, W11, b11, root1, rb1,
           bn_g2, bn_b2, W20, b20, W21, b21, root2, rb2, Wp, bp):
    e = e.reshape(-1, _EDGE_IN)
    n_edges = e.shape[0]
    n_out = Wp.shape[1]
    et = e.T
    wt = Wp.T
    bias = bp.reshape(n_out, 1)

    out_t = pl.pallas_call(
        _edge_pred_kernel,
        in_specs=[
            pl.BlockSpec(memory_space=pltpu.VMEM),
            pl.BlockSpec(memory_space=pltpu.VMEM),
            pl.BlockSpec(memory_space=pltpu.VMEM),
        ],
        out_specs=pl.BlockSpec(memory_space=pl.ANY),
        out_shape=jax.ShapeDtypeStruct((n_out, n_edges), jnp.float32),
        scratch_shapes=[
            pltpu.VMEM((n_out, n_edges), jnp.float32),
            pltpu.SemaphoreType.DMA((_CHUNKS,)),
        ],
    )(wt, bias, et)
    return out_t.T
